# baseline (speedup 1.0000x reference)
import functools

import jax
import jax.numpy as jnp
from jax import lax
from jax.experimental import pallas as pl
from jax.experimental.pallas import tpu as pltpu
from jax.experimental.pallas import tpu_sc as plsc

NBUF = 2


def _gather_flat(obs_flat, table, n_workers, chunk):
    n = obs_flat.shape[0]
    d = table.shape[1]
    per_w = n // n_workers
    steps = per_w // chunk
    assert steps % NBUF == 0
    mesh = plsc.VectorSubcoreMesh(core_axis_name="c", subcore_axis_name="s")

    @functools.partial(
        pl.kernel,
        mesh=mesh,
        out_type=jax.ShapeDtypeStruct((n, d), jnp.float32),
        scratch_types=[
            pltpu.VMEM((per_w,), jnp.int32),
            pltpu.VMEM((NBUF, chunk, d), jnp.float32),
        ]
        + [pltpu.SemaphoreType.DMA] * NBUF
        + [pltpu.SemaphoreType.DMA],
        compiler_params=pltpu.CompilerParams(use_tc_tiling_on_sc=False),
    )
    def run(obs_hbm, table_hbm, out_hbm, idx_v, rows_v, *sems):
        gat = sems[:NBUF]
        out_sem = sems[NBUF]
        info = plsc.get_sparse_core_info()
        nc = info.num_cores
        wid = lax.axis_index("s") * nc + lax.axis_index("c")
        wbase = wid * per_w

        active = wid < n_workers

        @pl.when(active)
        def _():
            pltpu.sync_copy(obs_hbm.at[pl.ds(wbase, per_w)], idx_v)

        def start_gather(g, b):
            pltpu.make_async_copy(
                table_hbm.at[idx_v.at[pl.ds(g * chunk, chunk)]],
                rows_v.at[b],
                gat[b],
            ).start()

        def wait_gather(g, b):
            pltpu.make_async_copy(
                table_hbm.at[idx_v.at[pl.ds(g * chunk, chunk)]],
                rows_v.at[b],
                gat[b],
            ).wait()

        def body(k, carry):
            g0 = NBUF * k
            for b in range(NBUF):
                start_gather(g0 + b, b)
            for b in range(NBUF):
                wait_gather(g0 + b, b)
            return carry

        @pl.when(active)
        def _():
            lax.fori_loop(0, steps // NBUF, body, 0)

            pltpu.make_async_copy(
                rows_v.at[0], out_hbm.at[pl.ds(wbase, chunk)], out_sem
            ).start()
            pltpu.make_async_copy(
                rows_v.at[0], out_hbm.at[pl.ds(wbase, chunk)], out_sem
            ).wait()

    return run(obs_flat, table)


def kernel(obs, table):
    b, f = obs.shape
    d = table.shape[1]
    n = b * f
    obs_flat = obs.reshape(n).astype(jnp.int32)
    out = _gather_flat(obs_flat, table, n_workers=16, chunk=256)
    return out.reshape(b, f * d)


# speedup vs baseline: 1.1026x; 1.1026x over previous
import functools

import jax
import jax.numpy as jnp
from jax import lax
from jax.experimental import pallas as pl
from jax.experimental.pallas import tpu as pltpu
from jax.experimental.pallas import tpu_sc as plsc

_NBUF = 4


def _gather_flat(obs_flat, table, n_workers, chunk):
    n = obs_flat.shape[0]
    d = table.shape[1]
    per_w = n // n_workers
    steps = per_w // chunk
    assert steps % _NBUF == 0 and steps >= 2 * _NBUF
    mesh = plsc.VectorSubcoreMesh(core_axis_name="c", subcore_axis_name="s")

    @functools.partial(
        pl.kernel,
        mesh=mesh,
        out_type=jax.ShapeDtypeStruct((n, d), jnp.float32),
        scratch_types=[
            pltpu.VMEM((per_w,), jnp.int32),
            pltpu.VMEM((_NBUF, chunk, d), jnp.float32),
        ]
        + [pltpu.SemaphoreType.DMA] * (2 * _NBUF),
        compiler_params=pltpu.CompilerParams(use_tc_tiling_on_sc=False),
    )
    def run(obs_hbm, table_hbm, out_hbm, idx_v, rows_v, *sems):
        gat = sems[:_NBUF]
        out = sems[_NBUF:]
        info = plsc.get_sparse_core_info()
        nc = info.num_cores
        wid = lax.axis_index("s") * nc + lax.axis_index("c")
        wbase = wid * per_w

        pltpu.sync_copy(obs_hbm.at[pl.ds(wbase, per_w)], idx_v)

        def gather_cp(g, b):
            return pltpu.make_async_copy(
                table_hbm.at[idx_v.at[pl.ds(g * chunk, chunk)]],
                rows_v.at[b],
                gat[b],
            )

        def store_cp(g, b):
            return pltpu.make_async_copy(
                rows_v.at[b],
                out_hbm.at[pl.ds(wbase + g * chunk, chunk)],
                out[b],
            )

        for b in range(_NBUF):
            gather_cp(b, b).start()
        for b in range(_NBUF):
            gather_cp(b, b).wait()
            store_cp(b, b).start()

        def body(k, carry):
            g0 = _NBUF * k
            for b in range(_NBUF):
                store_cp(g0 + b - _NBUF, b).wait()
                gather_cp(g0 + b, b).start()
            for b in range(_NBUF):
                gather_cp(g0 + b, b).wait()
                store_cp(g0 + b, b).start()
            return carry

        lax.fori_loop(1, steps // _NBUF, body, 0)
        for b in range(_NBUF):
            store_cp(steps - _NBUF + b, b).wait()

    return run(obs_flat, table)


def kernel(obs, table):
    b, f = obs.shape
    d = table.shape[1]
    n = b * f
    obs_flat = obs.reshape(n).astype(jnp.int32)
    out = _gather_flat(obs_flat, table, n_workers=32, chunk=512)
    return out.reshape(b, f * d)
